# baseline (speedup 1.0000x reference)
import functools

import jax
import jax.numpy as jnp
from jax import lax
from jax.experimental import pallas as pl
from jax.experimental.pallas import tpu as pltpu
from jax.experimental.pallas import tpu_sc as plsc

N_NODES = 10000
N_PAD = 10240
D_IN = 128
D_OUT = 64
DH = 32
N_EDGES = 320000
NC, NS = 2, 16
NW = NC * NS
E_PER_W = 10240
CHUNK = 128
IB = CHUNK // 128
N_CHUNKS = E_PER_W // CHUNK
NB = 4
N_GROUPS = N_CHUNKS // NB
E_PAD = NW * E_PER_W
ROWS_PER_TILE = N_PAD // NS
RB = 512
N_RB = N_PAD // RB

def _deg_body(dst_hbm, degp_hbm, dst_v, deg_v):
    cid = lax.axis_index("c")
    tid = lax.axis_index("s")
    wid = cid * NS + tid
    zeros16 = jnp.zeros((16,), jnp.float32)
    ones16 = jnp.ones((16,), jnp.float32)

    def _zero(i, _):
        deg_v[pl.ds(i * 16, 16)] = zeros16
        return 0

    lax.fori_loop(0, N_PAD // 16, _zero, 0)
    pltpu.sync_copy(dst_hbm.at[wid], dst_v)

    def _count(j, _):
        idx = dst_v[pl.ds(j * 16, 16)]
        plsc.addupdate_scatter(deg_v, [idx], ones16)
        return 0

    lax.fori_loop(0, E_PER_W // 16, _count, 0)
    pltpu.sync_copy(deg_v, degp_hbm.at[wid])


def _hop_body(g_hbm, src_hbm, dst_hbm, part_hbm,
              src_v, dst_v, rows_v, zbuf_v, g_sh, acc_sh, *sems):
    g_sems, s_sems = sems[:NB], sems[NB:]
    cid = lax.axis_index("c")
    tid = lax.axis_index("s")
    wid = cid * NS + tid
    row0 = tid * ROWS_PER_TILE
    zeros16 = jnp.zeros((16,), jnp.float32)

    def _zrow(r, _):
        for k in range(DH // 16):
            zbuf_v[r, pl.ds(k * 16, 16)] = zeros16
        return 0

    lax.fori_loop(0, CHUNK, _zrow, 0)
    pltpu.sync_copy(src_hbm.at[wid], src_v)
    pltpu.sync_copy(dst_hbm.at[wid], dst_v)

    def _gather(b, c):
        return pltpu.make_async_copy(
            g_sh.at[src_v.at[pl.ds(c * CHUNK, CHUNK)]], rows_v.at[b],
            g_sems[b])

    def _scatter_start(b, c):
        pltpu.async_copy(
            rows_v.at[b], acc_sh.at[dst_v.at[pl.ds(c * CHUNK, CHUNK)]],
            s_sems[b], add=True)

    def _scatter_wait(b, c):
        pltpu.make_async_copy(
            rows_v.at[b], acc_sh.at[dst_v.at[pl.ds(c * CHUNK, CHUNK)]],
            s_sems[b]).wait()

    for h in range(2):
        for j in range(ROWS_PER_TILE // CHUNK):
            pltpu.sync_copy(zbuf_v, acc_sh.at[pl.ds(row0 + j * CHUNK, CHUNK)])
        pltpu.sync_copy(g_hbm.at[h, pl.ds(row0, ROWS_PER_TILE)],
                        g_sh.at[pl.ds(row0, ROWS_PER_TILE)])
        plsc.subcore_barrier()

        for b in range(NB):
            _gather(b, b).start()

        def _group(gi, _):
            base = gi * NB
            for b in range(NB):
                _gather(b, base + b).wait()
                _scatter_start(b, base + b)
            for b in range(NB):
                _scatter_wait(b, base + b)
                _gather(b, base + b + NB).start()
            return 0

        lax.fori_loop(0, N_GROUPS - 1, _group, 0)
        base = (N_GROUPS - 1) * NB
        for b in range(NB):
            _gather(b, base + b).wait()
            _scatter_start(b, base + b)
        for b in range(NB):
            _scatter_wait(b, base + b)
        plsc.subcore_barrier()
        pltpu.sync_copy(acc_sh.at[pl.ds(row0, ROWS_PER_TILE)],
                        part_hbm.at[cid, h, pl.ds(row0, ROWS_PER_TILE)])


@functools.lru_cache(maxsize=None)
def _sc_kernels():
    mesh = plsc.VectorSubcoreMesh(
        core_axis_name="c", subcore_axis_name="s",
        num_cores=NC, num_subcores=NS)
    params = pltpu.CompilerParams(
        needs_layout_passes=False, use_tc_tiling_on_sc=False)
    deg_k = pl.kernel(
        _deg_body,
        out_type=jax.ShapeDtypeStruct((NW, N_PAD), jnp.float32),
        mesh=mesh,
        compiler_params=params,
        scratch_types=[
            pltpu.VMEM((E_PER_W,), jnp.int32),
            pltpu.VMEM((N_PAD,), jnp.float32),
        ],
    )
    hop_k = pl.kernel(
        _hop_body,
        out_type=jax.ShapeDtypeStruct((NC, 2, N_PAD, DH), jnp.float32),
        mesh=mesh,
        compiler_params=params,
        scratch_types=[
            pltpu.VMEM((E_PER_W,), jnp.int32),
            pltpu.VMEM((E_PER_W,), jnp.int32),
            pltpu.VMEM((NB, CHUNK, DH), jnp.float32),
            pltpu.VMEM((CHUNK, DH), jnp.float32),
            pltpu.VMEM_SHARED((N_PAD, DH), jnp.float32),
            pltpu.VMEM_SHARED((N_PAD, DH), jnp.float32),
        ] + [pltpu.SemaphoreType.DMA] * (2 * NB),
    )
    return deg_k, hop_k


def _mm_body(x_ref, wsg_ref, wfin_ref, degp_ref, g_ref, deg_ref):
    deg = jnp.sum(degp_ref[...], axis=0)
    nrm = lax.rsqrt(jnp.maximum(deg, 1.0))
    y = jnp.dot(x_ref[...], wsg_ref[...], preferred_element_type=jnp.float32)
    y = jnp.dot(y, wfin_ref[...], preferred_element_type=jnp.float32)
    g = y * nrm[:, None]
    g_ref[0] = g[:, :DH]
    g_ref[1] = g[:, DH:]
    deg_ref[...] = deg[:, None]


def _mid_body(part_ref, deg_ref, out_ref):
    s = part_ref[0] + part_ref[1]
    out_ref[...] = s / jnp.maximum(deg_ref[...], 1.0)[None]


def _fin_body(part_ref, deg_ref, b_ref, out_ref):
    s = part_ref[0] + part_ref[1]
    nrm = lax.rsqrt(jnp.maximum(deg_ref[...], 1.0))
    out_ref[...] = jnp.concatenate([s[0], s[1]], axis=1) * nrm + b_ref[...]


def kernel(x, edge_index, nonzer_index, nonzer_value, W_sg, W_fin, b_fin):
    del nonzer_index, nonzer_value
    src = edge_index[0].astype(jnp.int32)
    dst = edge_index[1].astype(jnp.int32)
    pad = E_PAD - N_EDGES
    src_p = jnp.concatenate(
        [src, N_NODES + (jnp.arange(pad, dtype=jnp.int32) % (N_PAD - N_NODES))])
    dst_p = jnp.concatenate(
        [dst, jnp.arange(pad, dtype=jnp.int32) % N_PAD])
    src3 = src_p.reshape(NW, E_PER_W)
    dst3 = dst_p.reshape(NW, E_PER_W)
    dst_deg = jnp.concatenate(
        [dst, N_NODES + (jnp.arange(pad, dtype=jnp.int32) % (N_PAD - N_NODES))])
    dst_flat = dst_deg.reshape(NW, E_PER_W)
    x_p = jnp.pad(x, ((0, N_PAD - N_NODES), (0, 0)))
    b2 = b_fin.reshape(1, D_OUT)

    _deg_kernel, _hop_kernel = _sc_kernels()
    degp = _deg_kernel(dst_flat)

    g, deg = pl.pallas_call(
        _mm_body,
        grid=(N_RB,),
        in_specs=[
            pl.BlockSpec((RB, D_IN), lambda i: (i, 0)),
            pl.BlockSpec((D_IN, D_IN), lambda i: (0, 0)),
            pl.BlockSpec((D_IN, D_OUT), lambda i: (0, 0)),
            pl.BlockSpec((NW, RB), lambda i: (0, i)),
        ],
        out_specs=[
            pl.BlockSpec((2, RB, DH), lambda i: (0, i, 0)),
            pl.BlockSpec((RB, 1), lambda i: (i, 0)),
        ],
        out_shape=[
            jax.ShapeDtypeStruct((2, N_PAD, DH), jnp.float32),
            jax.ShapeDtypeStruct((N_PAD, 1), jnp.float32),
        ],
    )(x_p, W_sg, W_fin, degp)

    part1 = _hop_kernel(g, src3, dst3)

    g2 = pl.pallas_call(
        _mid_body,
        grid=(N_RB,),
        in_specs=[
            pl.BlockSpec((NC, 2, RB, DH), lambda i: (0, 0, i, 0)),
            pl.BlockSpec((RB, 1), lambda i: (i, 0)),
        ],
        out_specs=pl.BlockSpec((2, RB, DH), lambda i: (0, i, 0)),
        out_shape=jax.ShapeDtypeStruct((2, N_PAD, DH), jnp.float32),
    )(part1, deg)

    part2 = _hop_kernel(g2, src3, dst3)

    out = pl.pallas_call(
        _fin_body,
        grid=(N_RB,),
        in_specs=[
            pl.BlockSpec((NC, 2, RB, DH), lambda i: (0, 0, i, 0)),
            pl.BlockSpec((RB, 1), lambda i: (i, 0)),
            pl.BlockSpec((1, D_OUT), lambda i: (0, 0)),
        ],
        out_specs=pl.BlockSpec((RB, D_OUT), lambda i: (i, 0)),
        out_shape=jax.ShapeDtypeStruct((N_PAD, D_OUT), jnp.float32),
    )(part2, deg, b2)

    return out[:N_NODES]


# speedup vs baseline: 1.3144x; 1.3144x over previous
import functools

import jax
import jax.numpy as jnp
from jax import lax
from jax.experimental import pallas as pl
from jax.experimental.pallas import tpu as pltpu
from jax.experimental.pallas import tpu_sc as plsc

N_NODES = 10000
N_PAD = 10240
D_IN = 128
D_OUT = 64
N_EDGES = 320000
NC, NS = 2, 16
NW = NC * NS
E_PER_W = 10240
CHUNK = 256
IB = CHUNK // 128
N_CHUNKS = E_PER_W // CHUNK
NB = 4
N_GROUPS = N_CHUNKS // NB
E_PAD = NW * E_PER_W
ROWS_PER_TILE = N_PAD // NS
RB = 512
N_RB = N_PAD // RB

def _deg_body(dst_hbm, degp_hbm, dst_v, deg_v):
    cid = lax.axis_index("c")
    tid = lax.axis_index("s")
    wid = cid * NS + tid
    zeros16 = jnp.zeros((16,), jnp.float32)
    ones16 = jnp.ones((16,), jnp.float32)

    def _zero(i, _):
        deg_v[pl.ds(i * 16, 16)] = zeros16
        return 0

    lax.fori_loop(0, N_PAD // 16, _zero, 0)
    pltpu.sync_copy(dst_hbm.at[wid], dst_v)

    def _count(j, _):
        idx = dst_v[pl.ds(j * 16, 16)]
        plsc.addupdate_scatter(deg_v, [idx], ones16)
        return 0

    lax.fori_loop(0, E_PER_W // 16, _count, 0)
    pltpu.sync_copy(deg_v, degp_hbm.at[wid])


def _hop_body(g_hbm, src_hbm, dst_hbm, zeros_hbm, part_hbm,
              src_v, dst_v, rows_v, acc_sh, *sems):
    g_sems, s_sems = sems[:NB], sems[NB:]
    cid = lax.axis_index("c")
    tid = lax.axis_index("s")
    wid = cid * NS + tid
    row0 = tid * ROWS_PER_TILE

    pltpu.sync_copy(zeros_hbm.at[pl.ds(row0, ROWS_PER_TILE)],
                    acc_sh.at[pl.ds(row0, ROWS_PER_TILE)])
    pltpu.sync_copy(src_hbm.at[wid], src_v)
    pltpu.sync_copy(dst_hbm.at[wid], dst_v)
    plsc.subcore_barrier()

    def _gather(b, c):
        return pltpu.make_async_copy(
            g_hbm.at[src_v.at[pl.ds(c * CHUNK, CHUNK)]], rows_v.at[b],
            g_sems[b])

    def _scatter_start(b, c):
        pltpu.async_copy(
            rows_v.at[b], acc_sh.at[dst_v.at[pl.ds(c * CHUNK, CHUNK)]],
            s_sems[b], add=True)

    def _scatter_wait(b, c):
        pltpu.make_async_copy(
            rows_v.at[b], acc_sh.at[dst_v.at[pl.ds(c * CHUNK, CHUNK)]],
            s_sems[b]).wait()

    for b in range(NB):
        _gather(b, b).start()

    def _group(gi, _):
        base = gi * NB
        for b in range(NB):
            _gather(b, base + b).wait()
            _scatter_start(b, base + b)
        for b in range(NB):
            _scatter_wait(b, base + b)
            _gather(b, base + b + NB).start()
        return 0

    lax.fori_loop(0, N_GROUPS - 1, _group, 0)
    base = (N_GROUPS - 1) * NB
    for b in range(NB):
        _gather(b, base + b).wait()
        _scatter_start(b, base + b)
    for b in range(NB):
        _scatter_wait(b, base + b)
    plsc.subcore_barrier()
    pltpu.sync_copy(acc_sh.at[pl.ds(row0, ROWS_PER_TILE)],
                    part_hbm.at[cid, pl.ds(row0, ROWS_PER_TILE)])


@functools.lru_cache(maxsize=None)
def _sc_kernels():
    mesh = plsc.VectorSubcoreMesh(
        core_axis_name="c", subcore_axis_name="s",
        num_cores=NC, num_subcores=NS)
    params = pltpu.CompilerParams(
        needs_layout_passes=False, use_tc_tiling_on_sc=False)
    deg_k = pl.kernel(
        _deg_body,
        out_type=jax.ShapeDtypeStruct((NW, N_PAD), jnp.float32),
        mesh=mesh,
        compiler_params=params,
        scratch_types=[
            pltpu.VMEM((E_PER_W,), jnp.int32),
            pltpu.VMEM((N_PAD,), jnp.float32),
        ],
    )
    hop_k = pl.kernel(
        _hop_body,
        out_type=jax.ShapeDtypeStruct((NC, N_PAD, D_OUT), jnp.float32),
        mesh=mesh,
        compiler_params=params,
        scratch_types=[
            pltpu.VMEM((E_PER_W,), jnp.int32),
            pltpu.VMEM((E_PER_W,), jnp.int32),
            pltpu.VMEM((NB, CHUNK, D_OUT), jnp.float32),
            pltpu.VMEM_SHARED((N_PAD, D_OUT), jnp.float32),
        ] + [pltpu.SemaphoreType.DMA] * (2 * NB),
    )
    return deg_k, hop_k


def _mm_body(x_ref, wsg_ref, wfin_ref, degp_ref, g_ref, deg_ref):
    deg = jnp.sum(degp_ref[...], axis=0)
    nrm = lax.rsqrt(jnp.maximum(deg, 1.0))
    y = jnp.dot(x_ref[...], wsg_ref[...], preferred_element_type=jnp.float32)
    y = jnp.dot(y, wfin_ref[...], preferred_element_type=jnp.float32)
    g_ref[...] = y * nrm[:, None]
    deg_ref[...] = deg[:, None]


def _mid_body(part_ref, deg_ref, out_ref):
    s = part_ref[0] + part_ref[1]
    out_ref[...] = s / jnp.maximum(deg_ref[...], 1.0)


def _fin_body(part_ref, deg_ref, b_ref, out_ref):
    s = part_ref[0] + part_ref[1]
    nrm = lax.rsqrt(jnp.maximum(deg_ref[...], 1.0))
    out_ref[...] = s * nrm + b_ref[...]


def kernel(x, edge_index, nonzer_index, nonzer_value, W_sg, W_fin, b_fin):
    del nonzer_index, nonzer_value
    src = edge_index[0].astype(jnp.int32)
    dst = edge_index[1].astype(jnp.int32)
    pad = E_PAD - N_EDGES
    src_p = jnp.concatenate(
        [src, N_NODES + (jnp.arange(pad, dtype=jnp.int32) % (N_PAD - N_NODES))])
    dst_p = jnp.concatenate(
        [dst, jnp.arange(pad, dtype=jnp.int32) % N_PAD])
    src3 = src_p.reshape(NW, E_PER_W)
    dst3 = dst_p.reshape(NW, E_PER_W)
    dst_deg = jnp.concatenate(
        [dst, N_NODES + (jnp.arange(pad, dtype=jnp.int32) % (N_PAD - N_NODES))])
    dst_flat = dst_deg.reshape(NW, E_PER_W)
    x_p = jnp.pad(x, ((0, N_PAD - N_NODES), (0, 0)))
    zeros_tbl = jnp.zeros((N_PAD, D_OUT), jnp.float32)
    b2 = b_fin.reshape(1, D_OUT)

    _deg_kernel, _hop_kernel = _sc_kernels()
    degp = _deg_kernel(dst_flat)

    g, deg = pl.pallas_call(
        _mm_body,
        grid=(N_RB,),
        in_specs=[
            pl.BlockSpec((RB, D_IN), lambda i: (i, 0)),
            pl.BlockSpec((D_IN, D_IN), lambda i: (0, 0)),
            pl.BlockSpec((D_IN, D_OUT), lambda i: (0, 0)),
            pl.BlockSpec((NW, RB), lambda i: (0, i)),
        ],
        out_specs=[
            pl.BlockSpec((RB, D_OUT), lambda i: (i, 0)),
            pl.BlockSpec((RB, 1), lambda i: (i, 0)),
        ],
        out_shape=[
            jax.ShapeDtypeStruct((N_PAD, D_OUT), jnp.float32),
            jax.ShapeDtypeStruct((N_PAD, 1), jnp.float32),
        ],
    )(x_p, W_sg, W_fin, degp)

    part1 = _hop_kernel(g, src3, dst3, zeros_tbl)

    g2 = pl.pallas_call(
        _mid_body,
        grid=(N_RB,),
        in_specs=[
            pl.BlockSpec((NC, RB, D_OUT), lambda i: (0, i, 0)),
            pl.BlockSpec((RB, 1), lambda i: (i, 0)),
        ],
        out_specs=pl.BlockSpec((RB, D_OUT), lambda i: (i, 0)),
        out_shape=jax.ShapeDtypeStruct((N_PAD, D_OUT), jnp.float32),
    )(part1, deg)

    part2 = _hop_kernel(g2, src3, dst3, zeros_tbl)

    out = pl.pallas_call(
        _fin_body,
        grid=(N_RB,),
        in_specs=[
            pl.BlockSpec((NC, RB, D_OUT), lambda i: (0, i, 0)),
            pl.BlockSpec((RB, 1), lambda i: (i, 0)),
            pl.BlockSpec((1, D_OUT), lambda i: (0, 0)),
        ],
        out_specs=pl.BlockSpec((RB, D_OUT), lambda i: (i, 0)),
        out_shape=jax.ShapeDtypeStruct((N_PAD, D_OUT), jnp.float32),
    )(part2, deg, b2)

    return out[:N_NODES]


# speedup vs baseline: 1.4403x; 1.0958x over previous
import functools

import jax
import jax.numpy as jnp
from jax import lax
from jax.experimental import pallas as pl
from jax.experimental.pallas import tpu as pltpu
from jax.experimental.pallas import tpu_sc as plsc

N_NODES = 10000
N_PAD = 10240
D_IN = 128
D_OUT = 64
N_EDGES = 320000
NC, NS = 2, 16
NW = NC * NS
E_PER_W = 10240
CHUNK = 256
IB = CHUNK // 128
N_CHUNKS = E_PER_W // CHUNK
NB = 4
N_GROUPS = N_CHUNKS // NB
E_PAD = NW * E_PER_W
ROWS_PER_TILE = N_PAD // NS
RB = 2048
N_RB = N_PAD // RB
RB_FIN = 1000
N_RB_FIN = N_NODES // RB_FIN

E_DEG_W = N_EDGES // NW


def _deg_body(dst_hbm, degp_hbm, dst_v, deg_v):
    cid = lax.axis_index("c")
    tid = lax.axis_index("s")
    wid = cid * NS + tid
    zeros16 = jnp.zeros((16,), jnp.float32)
    ones16 = jnp.ones((16,), jnp.float32)

    def _zero(i, _):
        deg_v[pl.ds(i * 16, 16)] = zeros16
        return 0

    lax.fori_loop(0, N_PAD // 16, _zero, 0)
    pltpu.sync_copy(dst_hbm.at[wid], dst_v)

    def _count(j, _):
        idx = dst_v[pl.ds(j * 16, 16)]
        plsc.addupdate_scatter(deg_v, [idx], ones16)
        return 0

    lax.fori_loop(0, E_DEG_W // 16, _count, 0)
    pltpu.sync_copy(deg_v, degp_hbm.at[wid])


def _hop_body(g_hbm, src_hbm, dst_hbm, zeros_hbm, part_hbm,
              src_v, dst_v, rows_v, acc_sh, *sems):
    g_sems, s_sems = sems[:NB], sems[NB:]
    cid = lax.axis_index("c")
    tid = lax.axis_index("s")
    wid = cid * NS + tid
    row0 = tid * ROWS_PER_TILE

    pltpu.sync_copy(zeros_hbm.at[pl.ds(row0, ROWS_PER_TILE)],
                    acc_sh.at[pl.ds(row0, ROWS_PER_TILE)])
    pltpu.sync_copy(src_hbm.at[wid], src_v)
    pltpu.sync_copy(dst_hbm.at[wid], dst_v)
    plsc.subcore_barrier()

    def _gather(b, c):
        return pltpu.make_async_copy(
            g_hbm.at[src_v.at[pl.ds(c * CHUNK, CHUNK)]], rows_v.at[b],
            g_sems[b])

    def _scatter_start(b, c):
        pltpu.async_copy(
            rows_v.at[b], acc_sh.at[dst_v.at[pl.ds(c * CHUNK, CHUNK)]],
            s_sems[b], add=True)

    def _scatter_wait(b, c):
        pltpu.make_async_copy(
            rows_v.at[b], acc_sh.at[dst_v.at[pl.ds(c * CHUNK, CHUNK)]],
            s_sems[b]).wait()

    for b in range(NB):
        _gather(b, b).start()

    def _group(gi, _):
        base = gi * NB
        for b in range(NB):
            _gather(b, base + b).wait()
            _scatter_start(b, base + b)
        for b in range(NB):
            _scatter_wait(b, base + b)
            _gather(b, base + b + NB).start()
        return 0

    lax.fori_loop(0, N_GROUPS - 1, _group, 0)
    base = (N_GROUPS - 1) * NB
    for b in range(NB):
        _gather(b, base + b).wait()
        _scatter_start(b, base + b)
    for b in range(NB):
        _scatter_wait(b, base + b)
    plsc.subcore_barrier()
    pltpu.sync_copy(acc_sh.at[pl.ds(row0, ROWS_PER_TILE)],
                    part_hbm.at[cid, pl.ds(row0, ROWS_PER_TILE)])


@functools.lru_cache(maxsize=None)
def _sc_kernels():
    mesh = plsc.VectorSubcoreMesh(
        core_axis_name="c", subcore_axis_name="s",
        num_cores=NC, num_subcores=NS)
    params = pltpu.CompilerParams(
        needs_layout_passes=False, use_tc_tiling_on_sc=False)
    deg_k = pl.kernel(
        _deg_body,
        out_type=jax.ShapeDtypeStruct((NW, N_PAD), jnp.float32),
        mesh=mesh,
        compiler_params=params,
        scratch_types=[
            pltpu.VMEM((E_DEG_W,), jnp.int32),
            pltpu.VMEM((N_PAD,), jnp.float32),
        ],
    )
    hop_k = pl.kernel(
        _hop_body,
        out_type=jax.ShapeDtypeStruct((NC, N_PAD, D_OUT), jnp.float32),
        mesh=mesh,
        compiler_params=params,
        scratch_types=[
            pltpu.VMEM((E_PER_W,), jnp.int32),
            pltpu.VMEM((E_PER_W,), jnp.int32),
            pltpu.VMEM((NB, CHUNK, D_OUT), jnp.float32),
            pltpu.VMEM_SHARED((N_PAD, D_OUT), jnp.float32),
        ] + [pltpu.SemaphoreType.DMA] * (2 * NB),
    )
    return deg_k, hop_k


def _mm_body(x_ref, wsg_ref, wfin_ref, degp_ref, g_ref, deg_ref):
    deg = jnp.sum(degp_ref[...], axis=0)
    nrm = lax.rsqrt(jnp.maximum(deg, 1.0))
    y = jnp.dot(x_ref[...], wsg_ref[...], preferred_element_type=jnp.float32)
    y = jnp.dot(y, wfin_ref[...], preferred_element_type=jnp.float32)
    row = pl.program_id(0) * RB + lax.broadcasted_iota(jnp.int32, (RB, 1), 0)
    g_ref[...] = jnp.where(row < N_NODES, y * nrm[:, None], 0.0)
    deg_ref[...] = deg[:, None]


def _mid_body(part_ref, deg_ref, out_ref):
    s = part_ref[0] + part_ref[1]
    out_ref[...] = s / jnp.maximum(deg_ref[...], 1.0)


def _fin_body(part_ref, deg_ref, b_ref, out_ref):
    s = part_ref[0] + part_ref[1]
    nrm = lax.rsqrt(jnp.maximum(deg_ref[...], 1.0))
    out_ref[...] = s * nrm + b_ref[...]


def kernel(x, edge_index, nonzer_index, nonzer_value, W_sg, W_fin, b_fin):
    del nonzer_index, nonzer_value
    src = edge_index[0].astype(jnp.int32)
    dst = edge_index[1].astype(jnp.int32)
    pad = E_PAD - N_EDGES
    src_p = jnp.concatenate(
        [src, N_NODES + (jnp.arange(pad, dtype=jnp.int32) % (N_PAD - N_NODES))])
    dst_p = jnp.concatenate(
        [dst, jnp.arange(pad, dtype=jnp.int32) % N_PAD])
    src3 = src_p.reshape(NW, E_PER_W)
    dst3 = dst_p.reshape(NW, E_PER_W)
    dst_flat = dst.reshape(NW, E_DEG_W)
    x_p = jnp.pad(x, ((0, N_PAD - N_NODES), (0, 0)))
    zeros_tbl = jnp.zeros((N_PAD, D_OUT), jnp.float32)
    b2 = b_fin.reshape(1, D_OUT)

    _deg_kernel, _hop_kernel = _sc_kernels()
    degp = _deg_kernel(dst_flat)

    g, deg = pl.pallas_call(
        _mm_body,
        grid=(N_RB,),
        in_specs=[
            pl.BlockSpec((RB, D_IN), lambda i: (i, 0)),
            pl.BlockSpec((D_IN, D_IN), lambda i: (0, 0)),
            pl.BlockSpec((D_IN, D_OUT), lambda i: (0, 0)),
            pl.BlockSpec((NW, RB), lambda i: (0, i)),
        ],
        out_specs=[
            pl.BlockSpec((RB, D_OUT), lambda i: (i, 0)),
            pl.BlockSpec((RB, 1), lambda i: (i, 0)),
        ],
        out_shape=[
            jax.ShapeDtypeStruct((N_PAD, D_OUT), jnp.float32),
            jax.ShapeDtypeStruct((N_PAD, 1), jnp.float32),
        ],
    )(x_p, W_sg, W_fin, degp)

    part1 = _hop_kernel(g, src3, dst3, zeros_tbl)

    g2 = pl.pallas_call(
        _mid_body,
        grid=(N_RB,),
        in_specs=[
            pl.BlockSpec((NC, RB, D_OUT), lambda i: (0, i, 0)),
            pl.BlockSpec((RB, 1), lambda i: (i, 0)),
        ],
        out_specs=pl.BlockSpec((RB, D_OUT), lambda i: (i, 0)),
        out_shape=jax.ShapeDtypeStruct((N_PAD, D_OUT), jnp.float32),
    )(part1, deg)

    part2 = _hop_kernel(g2, src3, dst3, zeros_tbl)

    out = pl.pallas_call(
        _fin_body,
        grid=(N_RB_FIN,),
        in_specs=[
            pl.BlockSpec((NC, RB_FIN, D_OUT), lambda i: (0, i, 0)),
            pl.BlockSpec((RB_FIN, 1), lambda i: (i, 0)),
            pl.BlockSpec((1, D_OUT), lambda i: (0, 0)),
        ],
        out_specs=pl.BlockSpec((RB_FIN, D_OUT), lambda i: (i, 0)),
        out_shape=jax.ShapeDtypeStruct((N_NODES, D_OUT), jnp.float32),
    )(part2, deg, b2)

    return out
